# Initial kernel scaffold; baseline (speedup 1.0000x reference)
import jax
import jax.numpy as jnp
from jax.experimental import pallas as pl


def kernel(x, edge_index, edge_attr, params):
    raise NotImplementedError("write your pallas kernel here")


# speedup vs baseline: 65.5911x; 65.5911x over previous
import functools
import numpy as np
import jax
import jax.numpy as jnp
from jax.experimental import pallas as pl

B = 128
T = 256
N = 17
E = 38
H = 16
C = N * H
G = B * T
GB = 256

BF = jnp.bfloat16
F32 = jnp.float32


def _k1_body(ei_ref, x_ref, w_ref, b_ref, r_ref, out_ref):
    src = ei_ref[:, :E]
    dst = ei_ref[:, E:]
    kidx = src * N + dst
    iota = jax.lax.broadcasted_iota(jnp.int32, (1, N * N), 1)
    s2 = jnp.zeros((GB, N * N), F32)
    for e in range(E):
        s2 = s2 + (kidx[:, e:e + 1] == iota).astype(F32)
    s2b = s2.astype(BF)
    rmat = r_ref[...]
    se = [
        jnp.dot(s2b[:, N * j:N * (j + 1)], rmat, preferred_element_type=F32)
        for j in range(N)
    ]
    h = x_ref[...]
    for l in range(3):
        z = h
        for j in range(N):
            z = z + se[j] * jnp.tile(h[:, H * j:H * (j + 1)], (1, N))
        y = jnp.maximum(jnp.dot(z, w_ref[2 * l]) + b_ref[2 * l], 0.0)
        y = jnp.dot(y, w_ref[2 * l + 1]) + b_ref[2 * l + 1]
        h = jnp.maximum(y, 0.0) if l < 2 else y
    out_ref[...] = h


def _k2_body(h_ref, w_ref, bias_ref, o_ref):
    h = h_ref[...]
    z = h + jnp.concatenate([jnp.zeros((1, C), F32), h[:-1, :]], axis=0)
    y = jnp.maximum(jnp.dot(z, w_ref[0:C, :H]) + bias_ref[0:1, :H], 0.0)
    y = jnp.dot(y, w_ref[C:C + H, :H]) + bias_ref[1:2, :H]
    h1 = jnp.maximum(y, 0.0)
    z1 = h1 + jnp.concatenate([jnp.zeros((1, H), F32), h1[:-1, :]], axis=0)
    y = jnp.maximum(jnp.dot(z1, w_ref[C + H:C + 2 * H, :H])
                    + bias_ref[2:3, :H], 0.0)
    y = jnp.dot(y, w_ref[C + 2 * H:C + 3 * H, :H]) + bias_ref[3:4, :H]
    o_ref[...] = y


def _bn(z, g, b):
    mu = jnp.mean(z, axis=0)
    var = jnp.var(z, axis=0)
    return (z - mu) / jnp.sqrt(var + 1e-5) * g + b


def _k3_body(h_ref, f1_ref, w3_ref, aux_ref, y1_ref, y2_ref, y3_ref):
    y = jnp.dot(h_ref[...], f1_ref[...]) + aux_ref[0:1, :]
    y = jnp.maximum(_bn(y, aux_ref[1:2, :], aux_ref[2:3, :]), 0.0)
    y = jnp.dot(y, w3_ref[0:64, :H]) + aux_ref[5:6, :H]
    y = _bn(y, aux_ref[3:4, :H], aux_ref[4:5, :H])
    yr = jnp.maximum(y, 0.0)
    y1_ref[...] = jnp.dot(yr, w3_ref[64:80, 0:3]) + aux_ref[6:7, 0:3]
    y2_ref[...] = jnp.dot(yr, w3_ref[64:80, 3:6]) + aux_ref[7:8, 0:3]
    y3_ref[...] = jnp.dot(yr, w3_ref[64:80, 6:16]) + aux_ref[8:9, 0:10]


def kernel(x, edge_index, edge_attr, params):
    del edge_attr

    xp = jnp.pad(x.astype(F32), ((0, 0), (0, 0), (0, 0), (0, H - 2)))
    x2 = xp.reshape(G, C)
    ei2 = edge_index.astype(jnp.int32).reshape(G, 2 * E)

    eyeN = jnp.eye(N, dtype=F32)
    rmat = jnp.repeat(eyeN, H, axis=1).astype(BF)

    wbd, bts = [], []
    for l in range(3):
        w1 = params['kp%d_W1' % l]
        if w1.shape[0] < H:
            w1 = jnp.pad(w1, ((0, H - w1.shape[0]), (0, 0)))
        wbd.append(jnp.kron(eyeN, w1.astype(F32)))
        wbd.append(jnp.kron(eyeN, params['kp%d_W2' % l].astype(F32)))
        bts.append(jnp.tile(params['kp%d_b1' % l].astype(F32), N))
        bts.append(jnp.tile(params['kp%d_b2' % l].astype(F32), N))
    wstack = jnp.stack(wbd)
    bstack = jnp.stack(bts)

    nblk = G // GB
    h_frames = pl.pallas_call(
        _k1_body,
        grid=(nblk,),
        in_specs=[
            pl.BlockSpec((GB, 2 * E), lambda i: (i, 0)),
            pl.BlockSpec((GB, C), lambda i: (i, 0)),
            pl.BlockSpec((6, C, C), lambda i: (0, 0, 0)),
            pl.BlockSpec((6, C), lambda i: (0, 0)),
            pl.BlockSpec((N, C), lambda i: (0, 0)),
        ],
        out_specs=pl.BlockSpec((GB, C), lambda i: (i, 0)),
        out_shape=jax.ShapeDtypeStruct((G, C), F32),
    )(ei2, x2, wstack, bstack, rmat)

    pad16 = lambda w: jnp.pad(w.astype(F32), ((0, 0), (0, C - w.shape[1])))
    brow = lambda b: jnp.pad(b.astype(F32), (0, C - b.shape[0]))[None, :]
    w2w = jnp.concatenate([
        pad16(params['tm0_W1']), pad16(params['tm0_W2']),
        pad16(params['tm1_W1']), pad16(params['tm1_W2']),
    ], axis=0)
    b2pack = jnp.concatenate([
        brow(params['tm0_b1']), brow(params['tm0_b2']),
        brow(params['tm1_b1']), brow(params['tm1_b2']),
    ], axis=0)

    ht = pl.pallas_call(
        _k2_body,
        grid=(B,),
        in_specs=[
            pl.BlockSpec((T, C), lambda b: (b, 0)),
            pl.BlockSpec(w2w.shape, lambda b: (0, 0)),
            pl.BlockSpec((4, C), lambda b: (0, 0)),
        ],
        out_specs=pl.BlockSpec((T, H), lambda b: (b, 0)),
        out_shape=jax.ShapeDtypeStruct((B * T, H), F32),
    )(h_frames.reshape(B * T, C), w2w, b2pack)

    hflat = ht.reshape(B, H * T)
    heads_w = jnp.concatenate([params['int_W'].astype(F32),
                               params['att_W'].astype(F32),
                               params['act_W'].astype(F32)], axis=1)
    w3 = jnp.concatenate([
        jnp.pad(params['fc2_W'].astype(F32), ((0, 0), (0, 64 - H))),
        jnp.pad(heads_w, ((0, 0), (0, 64 - H))),
    ], axis=0)
    pad64 = lambda a: jnp.pad(a.astype(F32), (0, 64 - a.shape[0]))[None, :]
    aux = jnp.concatenate([
        pad64(params['fc1_b']), pad64(params['bn1_g']), pad64(params['bn1_b']),
        pad64(params['bn2_g']), pad64(params['bn2_b']), pad64(params['fc2_b']),
        pad64(params['int_b']), pad64(params['att_b']), pad64(params['act_b']),
    ], axis=0)

    y1, y2, y3 = pl.pallas_call(
        _k3_body,
        in_specs=[
            pl.BlockSpec((B, H * T), lambda: (0, 0)),
            pl.BlockSpec((H * T, 64), lambda: (0, 0)),
            pl.BlockSpec((80, 64), lambda: (0, 0)),
            pl.BlockSpec((9, 64), lambda: (0, 0)),
        ],
        out_specs=[
            pl.BlockSpec((B, 3), lambda: (0, 0)),
            pl.BlockSpec((B, 3), lambda: (0, 0)),
            pl.BlockSpec((B, 10), lambda: (0, 0)),
        ],
        out_shape=[
            jax.ShapeDtypeStruct((B, 3), F32),
            jax.ShapeDtypeStruct((B, 3), F32),
            jax.ShapeDtypeStruct((B, 10), F32),
        ],
    )(hflat, params['fc1_W'].astype(F32), w3, aux)
    return (y1, y2, y3)


# speedup vs baseline: 68.4091x; 1.0430x over previous
import functools
import numpy as np
import jax
import jax.numpy as jnp
from jax.experimental import pallas as pl

B = 128
T = 256
N = 17
E = 38
H = 16
C = N * H
G = B * T
GB = 256

BF = jnp.bfloat16
F32 = jnp.float32

from jax import lax
from jax.experimental.pallas import tpu as pltpu, tpu_sc as plsc

ROW = 2 * E
SBIN = 304
NW = 32
GPW = G // NW
CH = 64
NCHUNK = GPW // CH


def _sc_adj_body(ei_hbm, s2_hbm, ei_v, s_v):
    wid = lax.axis_index("s") * 2 + lax.axis_index("c")
    gbase = wid * GPW
    zeros16 = jnp.zeros((16,), F32)
    ones16 = jnp.ones((16,), F32)
    m6 = lax.iota(jnp.int32, 16) < 6

    def chunk_body(ci, carry):
        cbase = gbase + ci * CH
        pltpu.sync_copy(ei_hbm.at[pl.ds(cbase * ROW, CH * ROW + 16)], ei_v)

        def zbody(i, c):
            s_v[pl.ds(i * 16, 16)] = zeros16
            return c
        lax.fori_loop(0, CH * SBIN // 16, zbody, 0)

        def gbody(g, c):
            ro = g * ROW
            gb = g * SBIN
            for off in (0, 16, 32):
                srcv = ei_v[pl.ds(ro + off, 16)]
                dstv = ei_v[pl.ds(ro + E + off, 16)]
                k = gb + srcv * N + dstv
                if off == 32:
                    plsc.addupdate_scatter(s_v, [k], ones16, mask=m6)
                else:
                    plsc.addupdate_scatter(s_v, [k], ones16)
            return c
        lax.fori_loop(0, CH, gbody, 0)
        pltpu.sync_copy(s_v, s2_hbm.at[pl.ds(cbase * SBIN, CH * SBIN)])
        return carry

    lax.fori_loop(0, NCHUNK, chunk_body, 0)


def _sc_adjacency(ei_flat_padded):
    mesh = plsc.VectorSubcoreMesh(core_axis_name="c", subcore_axis_name="s")
    f = pl.kernel(
        _sc_adj_body,
        out_type=jax.ShapeDtypeStruct((G * SBIN,), F32),
        mesh=mesh,
        scratch_types=[
            pltpu.VMEM((CH * ROW + 16,), jnp.int32),
            pltpu.VMEM((CH * SBIN,), F32),
        ],
        compiler_params=pltpu.CompilerParams(needs_layout_passes=False),
    )
    return f(ei_flat_padded)


def _k1_body(s2_ref, x_ref, w_ref, b_ref, r_ref, out_ref):
    s2b = s2_ref[...].astype(BF)
    rmat = r_ref[...]
    se = [
        jnp.dot(s2b[:, N * j:N * (j + 1)], rmat, preferred_element_type=F32)
        for j in range(N)
    ]
    h = x_ref[...]
    for l in range(3):
        z = h
        for j in range(N):
            z = z + se[j] * jnp.tile(h[:, H * j:H * (j + 1)], (1, N))
        y = jnp.maximum(jnp.dot(z, w_ref[2 * l]) + b_ref[2 * l], 0.0)
        y = jnp.dot(y, w_ref[2 * l + 1]) + b_ref[2 * l + 1]
        h = jnp.maximum(y, 0.0) if l < 2 else y
    out_ref[...] = h


def _k2_body(h_ref, w_ref, bias_ref, o_ref):
    h = h_ref[...]
    z = h + jnp.concatenate([jnp.zeros((1, C), F32), h[:-1, :]], axis=0)
    y = jnp.maximum(jnp.dot(z, w_ref[0:C, :H]) + bias_ref[0:1, :H], 0.0)
    y = jnp.dot(y, w_ref[C:C + H, :H]) + bias_ref[1:2, :H]
    h1 = jnp.maximum(y, 0.0)
    z1 = h1 + jnp.concatenate([jnp.zeros((1, H), F32), h1[:-1, :]], axis=0)
    y = jnp.maximum(jnp.dot(z1, w_ref[C + H:C + 2 * H, :H])
                    + bias_ref[2:3, :H], 0.0)
    y = jnp.dot(y, w_ref[C + 2 * H:C + 3 * H, :H]) + bias_ref[3:4, :H]
    o_ref[...] = y


def _bn(z, g, b):
    mu = jnp.mean(z, axis=0)
    var = jnp.var(z, axis=0)
    return (z - mu) / jnp.sqrt(var + 1e-5) * g + b


def _k3_body(h_ref, f1_ref, w3_ref, aux_ref, y1_ref, y2_ref, y3_ref):
    y = jnp.dot(h_ref[...], f1_ref[...]) + aux_ref[0:1, :]
    y = jnp.maximum(_bn(y, aux_ref[1:2, :], aux_ref[2:3, :]), 0.0)
    y = jnp.dot(y, w3_ref[0:64, :H]) + aux_ref[5:6, :H]
    y = _bn(y, aux_ref[3:4, :H], aux_ref[4:5, :H])
    yr = jnp.maximum(y, 0.0)
    y1_ref[...] = jnp.dot(yr, w3_ref[64:80, 0:3]) + aux_ref[6:7, 0:3]
    y2_ref[...] = jnp.dot(yr, w3_ref[64:80, 3:6]) + aux_ref[7:8, 0:3]
    y3_ref[...] = jnp.dot(yr, w3_ref[64:80, 6:16]) + aux_ref[8:9, 0:10]


def kernel(x, edge_index, edge_attr, params):
    del edge_attr

    xp = jnp.pad(x.astype(F32), ((0, 0), (0, 0), (0, 0), (0, H - 2)))
    x2 = xp.reshape(G, C)
    ei_flat = jnp.pad(edge_index.astype(jnp.int32).reshape(-1), (0, 16))
    s2 = _sc_adjacency(ei_flat).reshape(G, SBIN)

    eyeN = jnp.eye(N, dtype=F32)
    rmat = jnp.repeat(eyeN, H, axis=1).astype(BF)

    wbd, bts = [], []
    for l in range(3):
        w1 = params['kp%d_W1' % l]
        if w1.shape[0] < H:
            w1 = jnp.pad(w1, ((0, H - w1.shape[0]), (0, 0)))
        wbd.append(jnp.kron(eyeN, w1.astype(F32)))
        wbd.append(jnp.kron(eyeN, params['kp%d_W2' % l].astype(F32)))
        bts.append(jnp.tile(params['kp%d_b1' % l].astype(F32), N))
        bts.append(jnp.tile(params['kp%d_b2' % l].astype(F32), N))
    wstack = jnp.stack(wbd)
    bstack = jnp.stack(bts)

    nblk = G // GB
    h_frames = pl.pallas_call(
        _k1_body,
        grid=(nblk,),
        in_specs=[
            pl.BlockSpec((GB, SBIN), lambda i: (i, 0)),
            pl.BlockSpec((GB, C), lambda i: (i, 0)),
            pl.BlockSpec((6, C, C), lambda i: (0, 0, 0)),
            pl.BlockSpec((6, C), lambda i: (0, 0)),
            pl.BlockSpec((N, C), lambda i: (0, 0)),
        ],
        out_specs=pl.BlockSpec((GB, C), lambda i: (i, 0)),
        out_shape=jax.ShapeDtypeStruct((G, C), F32),
    )(s2, x2, wstack, bstack, rmat)

    pad16 = lambda w: jnp.pad(w.astype(F32), ((0, 0), (0, C - w.shape[1])))
    brow = lambda b: jnp.pad(b.astype(F32), (0, C - b.shape[0]))[None, :]
    w2w = jnp.concatenate([
        pad16(params['tm0_W1']), pad16(params['tm0_W2']),
        pad16(params['tm1_W1']), pad16(params['tm1_W2']),
    ], axis=0)
    b2pack = jnp.concatenate([
        brow(params['tm0_b1']), brow(params['tm0_b2']),
        brow(params['tm1_b1']), brow(params['tm1_b2']),
    ], axis=0)

    ht = pl.pallas_call(
        _k2_body,
        grid=(B,),
        in_specs=[
            pl.BlockSpec((T, C), lambda b: (b, 0)),
            pl.BlockSpec(w2w.shape, lambda b: (0, 0)),
            pl.BlockSpec((4, C), lambda b: (0, 0)),
        ],
        out_specs=pl.BlockSpec((T, H), lambda b: (b, 0)),
        out_shape=jax.ShapeDtypeStruct((B * T, H), F32),
    )(h_frames.reshape(B * T, C), w2w, b2pack)

    hflat = ht.reshape(B, H * T)
    heads_w = jnp.concatenate([params['int_W'].astype(F32),
                               params['att_W'].astype(F32),
                               params['act_W'].astype(F32)], axis=1)
    w3 = jnp.concatenate([
        jnp.pad(params['fc2_W'].astype(F32), ((0, 0), (0, 64 - H))),
        jnp.pad(heads_w, ((0, 0), (0, 64 - H))),
    ], axis=0)
    pad64 = lambda a: jnp.pad(a.astype(F32), (0, 64 - a.shape[0]))[None, :]
    aux = jnp.concatenate([
        pad64(params['fc1_b']), pad64(params['bn1_g']), pad64(params['bn1_b']),
        pad64(params['bn2_g']), pad64(params['bn2_b']), pad64(params['fc2_b']),
        pad64(params['int_b']), pad64(params['att_b']), pad64(params['act_b']),
    ], axis=0)

    y1, y2, y3 = pl.pallas_call(
        _k3_body,
        in_specs=[
            pl.BlockSpec((B, H * T), lambda: (0, 0)),
            pl.BlockSpec((H * T, 64), lambda: (0, 0)),
            pl.BlockSpec((80, 64), lambda: (0, 0)),
            pl.BlockSpec((9, 64), lambda: (0, 0)),
        ],
        out_specs=[
            pl.BlockSpec((B, 3), lambda: (0, 0)),
            pl.BlockSpec((B, 3), lambda: (0, 0)),
            pl.BlockSpec((B, 10), lambda: (0, 0)),
        ],
        out_shape=[
            jax.ShapeDtypeStruct((B, 3), F32),
            jax.ShapeDtypeStruct((B, 3), F32),
            jax.ShapeDtypeStruct((B, 10), F32),
        ],
    )(hflat, params['fc1_W'].astype(F32), w3, aux)
    return (y1, y2, y3)
